# initial kernel scaffold (unmeasured)
import jax
import jax.numpy as jnp
from jax import lax
from jax.experimental import pallas as pl
from jax.experimental.pallas import tpu as pltpu

N_DEV = 32
B = 2
SQ = 256
HQ = 4
DH = 64
BLK = 64
NBLK = SQ // BLK
DMODEL = 512
DQK = HQ * DH


def kernel(x, Wq, K_ext, V_ext, Wo):
    def body(x_ref, wq_ref, k_ref, v_ref, wo_ref, out_ref,
             kvbuf, ctx_ref, send_sems, recv_sems):
        my = lax.axis_index("i")
        left = lax.rem(my + N_DEV - 1, N_DEV)
        right = lax.rem(my + 1, N_DEV)

        barrier_sem = pltpu.get_barrier_semaphore()
        for nbr in (left, right):
            pl.semaphore_signal(barrier_sem, inc=1, device_id=(nbr,),
                                device_id_type=pl.DeviceIdType.MESH)
        pl.semaphore_wait(barrier_sem, 2)

        kvbuf[0, 0] = k_ref[...].astype(jnp.bfloat16)
        kvbuf[0, 1] = v_ref[...].astype(jnp.bfloat16)

        xm = x_ref[...].reshape(B * SQ, DMODEL).astype(jnp.bfloat16)
        wq = wq_ref[...].astype(jnp.bfloat16)
        q = lax.dot_general(xm, wq, (((1,), (0,)), ((), ())),
                            preferred_element_type=jnp.float32)
        q = (q * 0.125).reshape(B, SQ, HQ, DH).astype(jnp.bfloat16)

        for h in range(N_DEV - 1):
            rdma = pltpu.make_async_remote_copy(
                src_ref=kvbuf.at[h],
                dst_ref=kvbuf.at[h + 1],
                send_sem=send_sems.at[h],
                recv_sem=recv_sems.at[h],
                device_id=(right,),
                device_id_type=pl.DeviceIdType.MESH,
            )
            rdma.start()
            rdma.wait()

        for b in range(B):
            for r in range(NBLK):
                for hh in range(HQ):
                    qb = q[b, r * BLK:(r + 1) * BLK, hh, :]
                    kk = kvbuf[:, 0, b, r * BLK:(r + 1) * BLK, hh, :]
                    vv = kvbuf[:, 1, b, r * BLK:(r + 1) * BLK, hh, :]
                    kk = kk.reshape(N_DEV * BLK, DH)
                    vv = vv.reshape(N_DEV * BLK, DH)
                    s = lax.dot_general(qb, kk, (((1,), (1,)), ((), ())),
                                        preferred_element_type=jnp.float32)
                    m = jnp.max(s, axis=1, keepdims=True)
                    w = jnp.exp(s - m)
                    w = w / jnp.sum(w, axis=1, keepdims=True)
                    ctx = lax.dot_general(w.astype(jnp.bfloat16), vv,
                                          (((1,), (0,)), ((), ())),
                                          preferred_element_type=jnp.float32)
                    ctx_ref[b, r * BLK:(r + 1) * BLK, hh, :] = ctx

        c = ctx_ref[...].reshape(B * SQ, DQK).astype(jnp.bfloat16)
        wo = wo_ref[...].astype(jnp.bfloat16)
        o = lax.dot_general(c, wo, (((1,), (0,)), ((), ())),
                            preferred_element_type=jnp.float32)
        out_ref[...] = o.reshape(B, SQ, DMODEL)

    return pl.pallas_call(
        body,
        out_shape=jax.ShapeDtypeStruct((B, SQ, DMODEL), jnp.float32),
        in_specs=[pl.BlockSpec(memory_space=pltpu.VMEM)] * 5,
        out_specs=pl.BlockSpec(memory_space=pltpu.VMEM),
        scratch_shapes=[
            pltpu.VMEM((N_DEV, 2, B, SQ, HQ, DH), jnp.bfloat16),
            pltpu.VMEM((B, SQ, HQ, DH), jnp.float32),
            pltpu.SemaphoreType.DMA((N_DEV - 1,)),
            pltpu.SemaphoreType.DMA((N_DEV - 1,)),
        ],
        compiler_params=pltpu.CompilerParams(collective_id=0),
    )(x, Wq, K_ext, V_ext, Wo)


# baseline (device time: 429303 ns/iter reference)
import jax
import jax.numpy as jnp
from jax import lax
from jax.experimental import pallas as pl
from jax.experimental.pallas import tpu as pltpu

N_DEV = 32
B = 2
SQ = 256
HQ = 4
DH = 64
BLK = 64
NBLK = SQ // BLK
DMODEL = 512
DQK = HQ * DH
BH = B * HQ


def kernel(x, Wq, K_ext, V_ext, Wo):
    def body(x_ref, wq_ref, k_ref, v_ref, wo_ref, out_ref,
             kvbuf, ctx_ref, send_sems, recv_sems):
        my = lax.axis_index("i")
        left = lax.rem(my + N_DEV - 1, N_DEV)
        right = lax.rem(my + 1, N_DEV)

        barrier_sem = pltpu.get_barrier_semaphore()
        for nbr in (left, right):
            pl.semaphore_signal(barrier_sem, inc=1, device_id=(nbr,),
                                device_id_type=pl.DeviceIdType.MESH)
        pl.semaphore_wait(barrier_sem, 2)

        kvbuf[0, :, :, 0] = k_ref[...].astype(jnp.bfloat16).transpose(0, 2, 1, 3)
        kvbuf[1, :, :, 0] = v_ref[...].astype(jnp.bfloat16).transpose(0, 2, 1, 3)

        xm = x_ref[...].reshape(B * SQ, DMODEL).astype(jnp.bfloat16)
        wq = wq_ref[...].astype(jnp.bfloat16)
        q = lax.dot_general(xm, wq, (((1,), (0,)), ((), ())),
                            preferred_element_type=jnp.float32)
        q = (q * 0.125).reshape(B, SQ, HQ, DH).astype(jnp.bfloat16)
        q = q.transpose(0, 2, 1, 3).reshape(BH, SQ, DH)

        for h in range(N_DEV - 1):
            rdma = pltpu.make_async_remote_copy(
                src_ref=kvbuf.at[:, :, :, h],
                dst_ref=kvbuf.at[:, :, :, h + 1],
                send_sem=send_sems.at[h],
                recv_sem=recv_sems.at[h],
                device_id=(right,),
                device_id_type=pl.DeviceIdType.MESH,
            )
            rdma.start()
            rdma.wait()

        for r in range(NBLK):
            sl = pl.ds(r * BLK, BLK)
            qr = q[:, r * BLK:(r + 1) * BLK, :]
            kk = kvbuf[0, :, :, :, sl, :].reshape(BH, N_DEV * BLK, DH)
            vv = kvbuf[1, :, :, :, sl, :].reshape(BH, N_DEV * BLK, DH)
            s = lax.dot_general(qr, kk, (((2,), (2,)), ((0,), (0,))),
                                preferred_element_type=jnp.float32)
            m = jnp.max(s, axis=2, keepdims=True)
            w = jnp.exp(s - m)
            w = w / jnp.sum(w, axis=2, keepdims=True)
            ctx = lax.dot_general(w.astype(jnp.bfloat16), vv,
                                  (((2,), (1,)), ((0,), (0,))),
                                  preferred_element_type=jnp.float32)
            ctx_ref[:, :, sl, :] = ctx.reshape(B, HQ, BLK, DH)

        c = ctx_ref[...].astype(jnp.bfloat16).transpose(0, 2, 1, 3)
        c = c.reshape(B * SQ, DQK)
        wo = wo_ref[...].astype(jnp.bfloat16)
        o = lax.dot_general(c, wo, (((1,), (0,)), ((), ())),
                            preferred_element_type=jnp.float32)
        out_ref[...] = o.reshape(B, SQ, DMODEL)

    return pl.pallas_call(
        body,
        out_shape=jax.ShapeDtypeStruct((B, SQ, DMODEL), jnp.float32),
        in_specs=[pl.BlockSpec(memory_space=pltpu.VMEM)] * 5,
        out_specs=pl.BlockSpec(memory_space=pltpu.VMEM),
        scratch_shapes=[
            pltpu.VMEM((2, B, HQ, N_DEV, SQ, DH), jnp.bfloat16),
            pltpu.VMEM((B, HQ, SQ, DH), jnp.float32),
            pltpu.SemaphoreType.DMA((N_DEV - 1,)),
            pltpu.SemaphoreType.DMA((N_DEV - 1,)),
        ],
        compiler_params=pltpu.CompilerParams(
            collective_id=0,
            vmem_limit_bytes=64 * 1024 * 1024,
        ),
    )(x, Wq, K_ext, V_ext, Wo)


# device time: 378791 ns/iter; 1.1334x vs baseline; 1.1334x over previous
import jax
import jax.numpy as jnp
from jax import lax
from jax.experimental import pallas as pl
from jax.experimental.pallas import tpu as pltpu

N_DEV = 32
FWD = N_DEV // 2
BWD = N_DEV - 1 - FWD
B = 2
SQ = 256
HQ = 4
DH = 64
BLK = 64
NBLK = SQ // BLK
DMODEL = 512
DQK = HQ * DH
BH = B * HQ


def kernel(x, Wq, K_ext, V_ext, Wo):
    def body(x_ref, wq_ref, k_ref, v_ref, wo_ref, out_ref,
             kvbuf, ctx_ref, fsend, frecv, bsend, brecv):
        my = lax.axis_index("i")
        left = lax.rem(my + N_DEV - 1, N_DEV)
        right = lax.rem(my + 1, N_DEV)

        barrier_sem = pltpu.get_barrier_semaphore()
        for nbr in (left, right):
            pl.semaphore_signal(barrier_sem, inc=1, device_id=(nbr,),
                                device_id_type=pl.DeviceIdType.MESH)
        pl.semaphore_wait(barrier_sem, 2)

        kvbuf[0, :, :, 0] = k_ref[...].astype(jnp.bfloat16).transpose(0, 2, 1, 3)
        kvbuf[1, :, :, 0] = v_ref[...].astype(jnp.bfloat16).transpose(0, 2, 1, 3)

        def fwd_rdma(h):
            return pltpu.make_async_remote_copy(
                src_ref=kvbuf.at[:, :, :, h],
                dst_ref=kvbuf.at[:, :, :, h + 1],
                send_sem=fsend.at[h],
                recv_sem=frecv.at[h],
                device_id=(right,),
                device_id_type=pl.DeviceIdType.MESH,
            )

        def bwd_rdma(h):
            return pltpu.make_async_remote_copy(
                src_ref=kvbuf.at[:, :, :, 0 if h == 0 else FWD + h],
                dst_ref=kvbuf.at[:, :, :, FWD + 1 + h],
                send_sem=bsend.at[h],
                recv_sem=brecv.at[h],
                device_id=(left,),
                device_id_type=pl.DeviceIdType.MESH,
            )

        fw = [fwd_rdma(h) for h in range(FWD)]
        bw = [bwd_rdma(h) for h in range(BWD)]

        fw[0].start()
        bw[0].start()

        xm = x_ref[...].reshape(B * SQ, DMODEL).astype(jnp.bfloat16)
        wq = wq_ref[...].astype(jnp.bfloat16)
        q = lax.dot_general(xm, wq, (((1,), (0,)), ((), ())),
                            preferred_element_type=jnp.float32)
        q = (q * 0.125).reshape(B, SQ, HQ, DH).astype(jnp.bfloat16)
        q = q.transpose(0, 2, 1, 3).reshape(BH, SQ, DH)

        for h in range(1, FWD):
            fw[h - 1].wait_recv()
            fw[h].start()
            if h < BWD:
                bw[h - 1].wait_recv()
                bw[h].start()
        fw[FWD - 1].wait_recv()
        bw[BWD - 1].wait_recv()

        for r in range(NBLK):
            sl = pl.ds(r * BLK, BLK)
            qr = q[:, r * BLK:(r + 1) * BLK, :]
            kk = kvbuf[0, :, :, :, sl, :].reshape(BH, N_DEV * BLK, DH)
            vv = kvbuf[1, :, :, :, sl, :].reshape(BH, N_DEV * BLK, DH)
            s = lax.dot_general(qr, kk, (((2,), (2,)), ((0,), (0,))),
                                preferred_element_type=jnp.float32)
            m = jnp.max(s, axis=2, keepdims=True)
            w = jnp.exp(s - m)
            w = w / jnp.sum(w, axis=2, keepdims=True)
            ctx = lax.dot_general(w.astype(jnp.bfloat16), vv,
                                  (((2,), (1,)), ((0,), (0,))),
                                  preferred_element_type=jnp.float32)
            ctx_ref[:, :, sl, :] = ctx.reshape(B, HQ, BLK, DH)

        c = ctx_ref[...].astype(jnp.bfloat16).transpose(0, 2, 1, 3)
        c = c.reshape(B * SQ, DQK)
        wo = wo_ref[...].astype(jnp.bfloat16)
        o = lax.dot_general(c, wo, (((1,), (0,)), ((), ())),
                            preferred_element_type=jnp.float32)
        out_ref[...] = o.reshape(B, SQ, DMODEL)

        for r_ in fw + bw:
            r_.wait_send()

    return pl.pallas_call(
        body,
        out_shape=jax.ShapeDtypeStruct((B, SQ, DMODEL), jnp.float32),
        in_specs=[pl.BlockSpec(memory_space=pltpu.VMEM)] * 5,
        out_specs=pl.BlockSpec(memory_space=pltpu.VMEM),
        scratch_shapes=[
            pltpu.VMEM((2, B, HQ, N_DEV, SQ, DH), jnp.bfloat16),
            pltpu.VMEM((B, HQ, SQ, DH), jnp.float32),
            pltpu.SemaphoreType.DMA((FWD,)),
            pltpu.SemaphoreType.DMA((FWD,)),
            pltpu.SemaphoreType.DMA((BWD,)),
            pltpu.SemaphoreType.DMA((BWD,)),
        ],
        compiler_params=pltpu.CompilerParams(
            collective_id=0,
            vmem_limit_bytes=64 * 1024 * 1024,
        ),
    )(x, Wq, K_ext, V_ext, Wo)


# device time: 234034 ns/iter; 1.8344x vs baseline; 1.6185x over previous
import jax
import jax.numpy as jnp
from jax import lax
from jax.experimental import pallas as pl
from jax.experimental.pallas import tpu as pltpu

N_DEV = 32
FWD = N_DEV // 2
BWD = N_DEV - 1 - FWD
B = 2
SQ = 256
HQ = 4
DH = 64
BLK = 64
NBLK = SQ // BLK
DMODEL = 512
DQK = HQ * DH
BH = B * HQ

NEXT_TAB = (3, 0, 1, 4, 7, 2, 5, 15, 16, 10, 13, 8, 11, 14, 6, 12,
            19, 9, 17, 20, 23, 18, 21, 31, 25, 26, 29, 24, 27, 30, 22, 28)
PREV_TAB = (1, 2, 5, 0, 3, 6, 14, 4, 11, 17, 9, 12, 15, 10, 13, 7,
            8, 18, 21, 16, 19, 22, 30, 20, 27, 24, 25, 28, 31, 26, 29, 23)


def kernel(x, Wq, K_ext, V_ext, Wo):
    my = lax.axis_index("i")
    nxt = jnp.asarray(NEXT_TAB, dtype=jnp.int32)[my].reshape(1)
    prv = jnp.asarray(PREV_TAB, dtype=jnp.int32)[my].reshape(1)

    def body(x_ref, wq_ref, k_ref, v_ref, wo_ref, nxt_ref, prv_ref,
             out_ref, kvbuf, ctx_ref, fsend, frecv, bsend, brecv):
        right = nxt_ref[0]
        left = prv_ref[0]

        barrier_sem = pltpu.get_barrier_semaphore()
        for nbr in (left, right):
            pl.semaphore_signal(barrier_sem, inc=1, device_id=(nbr,),
                                device_id_type=pl.DeviceIdType.MESH)
        pl.semaphore_wait(barrier_sem, 2)

        kvbuf[0, :, :, 0] = k_ref[...].astype(jnp.bfloat16).transpose(0, 2, 1, 3)
        kvbuf[1, :, :, 0] = v_ref[...].astype(jnp.bfloat16).transpose(0, 2, 1, 3)

        def fwd_rdma(h):
            return pltpu.make_async_remote_copy(
                src_ref=kvbuf.at[:, :, :, h],
                dst_ref=kvbuf.at[:, :, :, h + 1],
                send_sem=fsend.at[h],
                recv_sem=frecv.at[h],
                device_id=(right,),
                device_id_type=pl.DeviceIdType.MESH,
            )

        def bwd_rdma(h):
            return pltpu.make_async_remote_copy(
                src_ref=kvbuf.at[:, :, :, 0 if h == 0 else FWD + h],
                dst_ref=kvbuf.at[:, :, :, FWD + 1 + h],
                send_sem=bsend.at[h],
                recv_sem=brecv.at[h],
                device_id=(left,),
                device_id_type=pl.DeviceIdType.MESH,
            )

        fw = [fwd_rdma(h) for h in range(FWD)]
        bw = [bwd_rdma(h) for h in range(BWD)]

        fw[0].start()
        bw[0].start()

        xm = x_ref[...].reshape(B * SQ, DMODEL).astype(jnp.bfloat16)
        wq = wq_ref[...].astype(jnp.bfloat16)
        q = lax.dot_general(xm, wq, (((1,), (0,)), ((), ())),
                            preferred_element_type=jnp.float32)
        q = (q * 0.125).reshape(B, SQ, HQ, DH).astype(jnp.bfloat16)
        q = q.transpose(0, 2, 1, 3).reshape(BH, SQ, DH)

        for h in range(1, FWD):
            fw[h - 1].wait_recv()
            fw[h].start()
            if h < BWD:
                bw[h - 1].wait_recv()
                bw[h].start()
        fw[FWD - 1].wait_recv()
        bw[BWD - 1].wait_recv()

        for r in range(NBLK):
            sl = pl.ds(r * BLK, BLK)
            qr = q[:, r * BLK:(r + 1) * BLK, :]
            kk = kvbuf[0, :, :, :, sl, :].reshape(BH, N_DEV * BLK, DH)
            vv = kvbuf[1, :, :, :, sl, :].reshape(BH, N_DEV * BLK, DH)
            s = lax.dot_general(qr, kk, (((2,), (2,)), ((0,), (0,))),
                                preferred_element_type=jnp.float32)
            m = jnp.max(s, axis=2, keepdims=True)
            w = jnp.exp(s - m)
            w = w / jnp.sum(w, axis=2, keepdims=True)
            ctx = lax.dot_general(w.astype(jnp.bfloat16), vv,
                                  (((2,), (1,)), ((0,), (0,))),
                                  preferred_element_type=jnp.float32)
            ctx_ref[:, :, sl, :] = ctx.reshape(B, HQ, BLK, DH)

        c = ctx_ref[...].astype(jnp.bfloat16).transpose(0, 2, 1, 3)
        c = c.reshape(B * SQ, DQK)
        wo = wo_ref[...].astype(jnp.bfloat16)
        o = lax.dot_general(c, wo, (((1,), (0,)), ((), ())),
                            preferred_element_type=jnp.float32)
        out_ref[...] = o.reshape(B, SQ, DMODEL)

        for r_ in fw + bw:
            r_.wait_send()

    return pl.pallas_call(
        body,
        out_shape=jax.ShapeDtypeStruct((B, SQ, DMODEL), jnp.float32),
        in_specs=[pl.BlockSpec(memory_space=pltpu.VMEM)] * 5
        + [pl.BlockSpec(memory_space=pltpu.SMEM)] * 2,
        out_specs=pl.BlockSpec(memory_space=pltpu.VMEM),
        scratch_shapes=[
            pltpu.VMEM((2, B, HQ, N_DEV, SQ, DH), jnp.bfloat16),
            pltpu.VMEM((B, HQ, SQ, DH), jnp.float32),
            pltpu.SemaphoreType.DMA((FWD,)),
            pltpu.SemaphoreType.DMA((FWD,)),
            pltpu.SemaphoreType.DMA((BWD,)),
            pltpu.SemaphoreType.DMA((BWD,)),
        ],
        compiler_params=pltpu.CompilerParams(
            collective_id=0,
            vmem_limit_bytes=64 * 1024 * 1024,
        ),
    )(x, Wq, K_ext, V_ext, Wo, nxt, prv)
